# baseline (speedup 1.0000x reference)
import jax
import jax.numpy as jnp
from jax import lax
from jax.experimental import pallas as pl
from jax.experimental.pallas import tpu as pltpu
from jax.experimental.pallas import tpu_sc as plsc

TOP_K = 8
N_EXPERTS = 256
SCALING = 2.5
N_GROUP = 8
TOPK_GROUP = 4
GROUP_SIZE = N_EXPERTS // N_GROUP

NC, NS, L = 2, 16, 16
NW = NC * NS
N_SUPER = N_EXPERTS // L
SUP_PER_GROUP = GROUP_SIZE // L
NEG = -1e9
TILE_ELEMS = L * N_EXPERTS


def _router_body(adj_hbm, bias_hbm, idx_hbm, w_hbm, xbuf, bias_v, oi, ow,
                 sem):
    n_tokens = adj_hbm.shape[0] // N_EXPERTS
    tok_per_w = n_tokens // NW
    n_tiles = tok_per_w // L

    wid = lax.axis_index("s") * NC + lax.axis_index("c")
    lane = lax.iota(jnp.int32, L)
    lane_row = lane * N_EXPERTS
    lane8 = lane * TOP_K
    negv = jnp.full((L,), NEG, jnp.float32)
    zero_i = jnp.zeros((L,), jnp.int32)

    pltpu.sync_copy(bias_hbm, bias_v)
    in_base = wid * tok_per_w * N_EXPERTS
    pltpu.async_copy(adj_hbm.at[pl.ds(in_base, TILE_ELEMS)],
                     xbuf.at[pl.ds(0, TILE_ELEMS)], sem)

    def tile_body(t, _):
        p = lax.rem(t, 2)
        pbase = p * TILE_ELEMS
        pltpu.make_async_copy(
            adj_hbm.at[pl.ds(in_base + t * TILE_ELEMS, TILE_ELEMS)],
            xbuf.at[pl.ds(pbase, TILE_ELEMS)], sem).wait()

        @pl.when(t < n_tiles - 1)
        def _prefetch():
            pltpu.async_copy(
                adj_hbm.at[pl.ds(in_base + (t + 1) * TILE_ELEMS, TILE_ELEMS)],
                xbuf.at[pl.ds(TILE_ELEMS - pbase, TILE_ELEMS)], sem)

        vb = pbase + lane_row

        sm1 = [negv] * N_SUPER
        sm2 = [negv] * N_SUPER
        for k in range(N_EXPERTS):
            si = k % N_SUPER
            e = si * L + k // N_SUPER
            s = plsc.load_gather(xbuf, [vb + e])
            gt = s > sm1[si]
            sm2[si] = jnp.where(gt, sm1[si], jnp.maximum(sm2[si], s))
            sm1[si] = jnp.maximum(sm1[si], s)

        g_sc = []
        for g in range(N_GROUP):
            a1, a2 = sm1[2 * g], sm2[2 * g]
            b1, b2 = sm1[2 * g + 1], sm2[2 * g + 1]
            hi = jnp.maximum(a1, b1)
            lo = jnp.minimum(a1, b1)
            sec = jnp.maximum(lo, jnp.where(a1 > b1, a2, b2))
            g_sc.append(hi + sec)

        chosen = [None] * N_GROUP
        for _ in range(TOPK_GROUP):
            best = g_sc[0]
            for g in range(1, N_GROUP):
                best = jnp.maximum(best, g_sc[g])
            found = lane < 0
            for g in range(N_GROUP):
                hit = (g_sc[g] == best) & (~found)
                chosen[g] = hit if chosen[g] is None else (chosen[g] | hit)
                found = found | hit
                g_sc[g] = jnp.where(hit, negv, g_sc[g])

        sup = [jnp.where(chosen[s_i // SUP_PER_GROUP], sm1[s_i], negv)
               for s_i in range(N_SUPER)]

        wsum = jnp.zeros((L,), jnp.float32)
        obase = t * L * TOP_K
        ws = []
        for r in range(TOP_K):
            m = negv
            mi = zero_i
            for s_i in range(N_SUPER):
                gt = sup[s_i] > m
                m = jnp.where(gt, sup[s_i], m)
                mi = jnp.where(gt, jnp.full((L,), s_i, jnp.int32), mi)
            lbase = vb + mi * L
            lm = negv
            lm2 = negv
            lj = zero_i
            for j in range(L):
                v = plsc.load_gather(xbuf, [lbase + j])
                gt = v > lm
                lm2 = jnp.where(gt, lm, jnp.maximum(lm2, v))
                lm = jnp.maximum(lm, v)
                lj = jnp.where(gt, jnp.full((L,), j, jnp.int32), lj)
            eidx = mi * L + lj
            plsc.store_scatter(xbuf, [lbase + lj], negv)
            hit_s = [mi == s_i for s_i in range(N_SUPER)]
            sup = [jnp.where(hit_s[s_i], lm2, sup[s_i])
                   for s_i in range(N_SUPER)]
            b = plsc.load_gather(bias_v, [eidx])
            w = lm - b
            plsc.store_scatter(oi, [lane8 + (obase + r)], eidx)
            ws.append(w)
            wsum = wsum + w

        scale = SCALING / (wsum + 1e-20)
        for r in range(TOP_K):
            plsc.store_scatter(ow, [lane8 + (obase + r)], ws[r] * scale)
        return 0

    lax.fori_loop(0, n_tiles, tile_body, 0)
    out_base = wid * tok_per_w * TOP_K
    out_len = tok_per_w * TOP_K
    pltpu.sync_copy(oi, idx_hbm.at[pl.ds(out_base, out_len)])
    pltpu.sync_copy(ow, w_hbm.at[pl.ds(out_base, out_len)])


def kernel(router_logits, correction_bias):
    scores = jax.nn.sigmoid(router_logits).astype(jnp.float32)
    adj = scores + correction_bias[None, :]
    n_tokens = router_logits.shape[0]
    tok_per_w = n_tokens // NW
    mesh = plsc.VectorSubcoreMesh(core_axis_name="c", subcore_axis_name="s",
                                  num_cores=NC, num_subcores=NS)
    run = pl.kernel(
        _router_body,
        out_type=(
            jax.ShapeDtypeStruct((n_tokens * TOP_K,), jnp.int32),
            jax.ShapeDtypeStruct((n_tokens * TOP_K,), jnp.float32),
        ),
        mesh=mesh,
        scratch_types=[
            pltpu.VMEM((2 * TILE_ELEMS,), jnp.float32),
            pltpu.VMEM((N_EXPERTS,), jnp.float32),
            pltpu.VMEM((tok_per_w * TOP_K,), jnp.int32),
            pltpu.VMEM((tok_per_w * TOP_K,), jnp.float32),
            pltpu.SemaphoreType.DMA,
        ],
        compiler_params=pltpu.CompilerParams(use_tc_tiling_on_sc=False,
                                             needs_layout_passes=False),
    )
    idx_flat, w_flat = run(adj.reshape(n_tokens * N_EXPERTS), correction_bias)
    return (idx_flat.reshape(n_tokens, TOP_K),
            w_flat.reshape(n_tokens, TOP_K))


# speedup vs baseline: 1.0363x; 1.0363x over previous
import jax
import jax.numpy as jnp
from jax import lax
from jax.experimental import pallas as pl
from jax.experimental.pallas import tpu as pltpu
from jax.experimental.pallas import tpu_sc as plsc

TOP_K = 8
N_EXPERTS = 256
SCALING = 2.5
N_GROUP = 8
TOPK_GROUP = 4
GROUP_SIZE = N_EXPERTS // N_GROUP

NC, NS, L = 2, 16, 16
NW = NC * NS
N_SUPER = N_EXPERTS // L
SUP_PER_GROUP = GROUP_SIZE // L
NEG = -1e9
TILE_ELEMS = L * N_EXPERTS


def _router_body(adj_hbm, bias_hbm, idx_hbm, w_hbm, xbuf, sup, bias_v, oi, ow,
                 sem):
    n_tokens = adj_hbm.shape[0] // N_EXPERTS
    tok_per_w = n_tokens // NW
    n_tiles = tok_per_w // L

    wid = lax.axis_index("s") * NC + lax.axis_index("c")
    lane = lax.iota(jnp.int32, L)
    lane_row = lane * N_EXPERTS
    lane8 = lane * TOP_K
    negv = jnp.full((L,), NEG, jnp.float32)
    zero_i = jnp.zeros((L,), jnp.int32)

    pltpu.sync_copy(bias_hbm, bias_v)
    in_base = wid * tok_per_w * N_EXPERTS
    pltpu.async_copy(adj_hbm.at[pl.ds(in_base, TILE_ELEMS)],
                     xbuf.at[pl.ds(0, TILE_ELEMS)], sem)

    def tile_body(t, _):
        p = lax.rem(t, 2)
        pbase = p * TILE_ELEMS
        pltpu.make_async_copy(
            adj_hbm.at[pl.ds(in_base + t * TILE_ELEMS, TILE_ELEMS)],
            xbuf.at[pl.ds(pbase, TILE_ELEMS)], sem).wait()

        @pl.when(t < n_tiles - 1)
        def _prefetch():
            pltpu.async_copy(
                adj_hbm.at[pl.ds(in_base + (t + 1) * TILE_ELEMS, TILE_ELEMS)],
                xbuf.at[pl.ds(TILE_ELEMS - pbase, TILE_ELEMS)], sem)

        vb = pbase + lane_row

        sm1 = [negv] * N_SUPER
        sm2 = [negv] * N_SUPER
        for k in range(N_EXPERTS):
            si = k % N_SUPER
            e = si * L + k // N_SUPER
            s = plsc.load_gather(xbuf, [vb + e])
            gt = s > sm1[si]
            sm2[si] = jnp.where(gt, sm1[si], jnp.maximum(sm2[si], s))
            sm1[si] = jnp.maximum(sm1[si], s)

        g_sc = []
        for g in range(N_GROUP):
            a1, a2 = sm1[2 * g], sm2[2 * g]
            b1, b2 = sm1[2 * g + 1], sm2[2 * g + 1]
            hi = jnp.maximum(a1, b1)
            lo = jnp.minimum(a1, b1)
            sec = jnp.maximum(lo, jnp.where(a1 > b1, a2, b2))
            g_sc.append(hi + sec)

        chosen = [None] * N_GROUP
        for _ in range(TOPK_GROUP):
            best = g_sc[0]
            for g in range(1, N_GROUP):
                best = jnp.maximum(best, g_sc[g])
            found = lane < 0
            for g in range(N_GROUP):
                hit = (g_sc[g] == best) & (~found)
                chosen[g] = hit if chosen[g] is None else (chosen[g] | hit)
                found = found | hit
                g_sc[g] = jnp.where(hit, negv, g_sc[g])

        for s_i in range(N_SUPER):
            sup[pl.ds(s_i * L, L)] = jnp.where(chosen[s_i // SUP_PER_GROUP],
                                               sm1[s_i], negv)

        wsum = jnp.zeros((L,), jnp.float32)
        obase = t * L * TOP_K
        ws = []
        for r in range(TOP_K):
            m = negv
            mi = zero_i
            for s_i in range(N_SUPER):
                v = sup[pl.ds(s_i * L, L)]
                gt = v > m
                m = jnp.where(gt, v, m)
                mi = jnp.where(gt, jnp.full((L,), s_i, jnp.int32), mi)
            lbase = vb + mi * L
            lm = negv
            lm2 = negv
            lj = zero_i
            for j in range(L):
                v = plsc.load_gather(xbuf, [lbase + j])
                gt = v > lm
                lm2 = jnp.where(gt, lm, jnp.maximum(lm2, v))
                lm = jnp.maximum(lm, v)
                lj = jnp.where(gt, jnp.full((L,), j, jnp.int32), lj)
            eidx = mi * L + lj
            plsc.store_scatter(xbuf, [lbase + lj], negv)
            plsc.store_scatter(sup, [mi * L + lane], lm2)
            b = plsc.load_gather(bias_v, [eidx])
            w = lm - b
            plsc.store_scatter(oi, [lane8 + (obase + r)], eidx)
            ws.append(w)
            wsum = wsum + w

        scale = SCALING / (wsum + 1e-20)
        for r in range(TOP_K):
            plsc.store_scatter(ow, [lane8 + (obase + r)], ws[r] * scale)
        return 0

    lax.fori_loop(0, n_tiles, tile_body, 0)
    out_base = wid * tok_per_w * TOP_K
    out_len = tok_per_w * TOP_K
    pltpu.sync_copy(oi, idx_hbm.at[pl.ds(out_base, out_len)])
    pltpu.sync_copy(ow, w_hbm.at[pl.ds(out_base, out_len)])


def kernel(router_logits, correction_bias):
    scores = jax.nn.sigmoid(router_logits).astype(jnp.float32)
    adj = scores + correction_bias[None, :]
    n_tokens = router_logits.shape[0]
    tok_per_w = n_tokens // NW
    mesh = plsc.VectorSubcoreMesh(core_axis_name="c", subcore_axis_name="s",
                                  num_cores=NC, num_subcores=NS)
    run = pl.kernel(
        _router_body,
        out_type=(
            jax.ShapeDtypeStruct((n_tokens * TOP_K,), jnp.int32),
            jax.ShapeDtypeStruct((n_tokens * TOP_K,), jnp.float32),
        ),
        mesh=mesh,
        scratch_types=[
            pltpu.VMEM((2 * TILE_ELEMS,), jnp.float32),
            pltpu.VMEM((N_SUPER * L,), jnp.float32),
            pltpu.VMEM((N_EXPERTS,), jnp.float32),
            pltpu.VMEM((tok_per_w * TOP_K,), jnp.int32),
            pltpu.VMEM((tok_per_w * TOP_K,), jnp.float32),
            pltpu.SemaphoreType.DMA,
        ],
        compiler_params=pltpu.CompilerParams(use_tc_tiling_on_sc=False,
                                             needs_layout_passes=False),
    )
    idx_flat, w_flat = run(adj.reshape(n_tokens * N_EXPERTS), correction_bias)
    return (idx_flat.reshape(n_tokens, TOP_K),
            w_flat.reshape(n_tokens, TOP_K))


# speedup vs baseline: 1.0367x; 1.0004x over previous
import jax
import jax.numpy as jnp
from jax import lax
from jax.experimental import pallas as pl
from jax.experimental.pallas import tpu as pltpu
from jax.experimental.pallas import tpu_sc as plsc

TOP_K = 8
N_EXPERTS = 256
SCALING = 2.5
N_GROUP = 8
TOPK_GROUP = 4
GROUP_SIZE = N_EXPERTS // N_GROUP

NC, NS, L = 2, 16, 16
NW = NC * NS
N_SUPER = N_EXPERTS // L
SUP_PER_GROUP = GROUP_SIZE // L
NEG = -1e9
TILE_ELEMS = L * N_EXPERTS


def _router_body(adj_hbm, bias_hbm, idx_hbm, w_hbm, xbuf, sup, bias_v, oi, ow,
                 sem):
    n_tokens = adj_hbm.shape[0] // N_EXPERTS
    tok_per_w = n_tokens // NW
    n_tiles = tok_per_w // L

    wid = lax.axis_index("s") * NC + lax.axis_index("c")
    lane = lax.iota(jnp.int32, L)
    lane_row = lane * N_EXPERTS
    lane8 = lane * TOP_K
    negv = jnp.full((L,), NEG, jnp.float32)
    zero_i = jnp.zeros((L,), jnp.int32)

    pltpu.sync_copy(bias_hbm, bias_v)
    in_base = wid * tok_per_w * N_EXPERTS
    pltpu.async_copy(adj_hbm.at[pl.ds(in_base, TILE_ELEMS)],
                     xbuf.at[pl.ds(0, TILE_ELEMS)], sem)

    def tile_body(t, _):
        p = lax.rem(t, 2)
        pbase = p * TILE_ELEMS
        pltpu.make_async_copy(
            adj_hbm.at[pl.ds(in_base + t * TILE_ELEMS, TILE_ELEMS)],
            xbuf.at[pl.ds(pbase, TILE_ELEMS)], sem).wait()

        @pl.when(t < n_tiles - 1)
        def _prefetch():
            pltpu.async_copy(
                adj_hbm.at[pl.ds(in_base + (t + 1) * TILE_ELEMS, TILE_ELEMS)],
                xbuf.at[pl.ds(TILE_ELEMS - pbase, TILE_ELEMS)], sem)

        vb = pbase + lane_row

        sm1 = [negv] * N_SUPER
        sm2 = [negv] * N_SUPER
        for k in range(N_EXPERTS):
            si = k % N_SUPER
            e = si * L + k // N_SUPER
            s = plsc.load_gather(xbuf, [vb + e])
            gt = s > sm1[si]
            sm2[si] = jnp.where(gt, sm1[si], jnp.maximum(sm2[si], s))
            sm1[si] = jnp.maximum(sm1[si], s)

        g_sc = []
        for g in range(N_GROUP):
            a1, a2 = sm1[2 * g], sm2[2 * g]
            b1, b2 = sm1[2 * g + 1], sm2[2 * g + 1]
            hi = jnp.maximum(a1, b1)
            lo = jnp.minimum(a1, b1)
            sec = jnp.maximum(lo, jnp.where(a1 > b1, a2, b2))
            g_sc.append(hi + sec)

        chosen = [None] * N_GROUP
        for _ in range(TOPK_GROUP):
            best = g_sc[0]
            for g in range(1, N_GROUP):
                best = jnp.maximum(best, g_sc[g])
            found = lane < 0
            for g in range(N_GROUP):
                hit = (g_sc[g] == best) & (~found)
                chosen[g] = hit if chosen[g] is None else (chosen[g] | hit)
                found = found | hit
                g_sc[g] = jnp.where(hit, negv, g_sc[g])

        for s_i in range(N_SUPER):
            sup[pl.ds(s_i * L, L)] = jnp.where(chosen[s_i // SUP_PER_GROUP],
                                               sm1[s_i], negv)

        wsum = jnp.zeros((L,), jnp.float32)
        obase = t * L * TOP_K
        ws = []
        for r in range(TOP_K):
            m = negv
            mi = zero_i
            for s_i in range(N_SUPER):
                v = sup[pl.ds(s_i * L, L)]
                gt = v > m
                m = jnp.where(gt, v, m)
                mi = jnp.where(gt, jnp.full((L,), s_i, jnp.int32), mi)
            lbase = vb + mi * L
            lm = negv
            lm2 = negv
            lj = zero_i
            for j in range(L):
                v = plsc.load_gather(xbuf, [lbase + j])
                gt = v > lm
                lm2 = jnp.where(gt, lm, jnp.maximum(lm2, v))
                lm = jnp.maximum(lm, v)
                lj = jnp.where(gt, jnp.full((L,), j, jnp.int32), lj)
            eidx = mi * L + lj
            plsc.store_scatter(xbuf, [lbase + lj], negv)
            plsc.store_scatter(sup, [mi * L + lane], lm2)
            b = plsc.load_gather(bias_v, [eidx])
            w = lm - b
            plsc.store_scatter(oi, [lane8 + (obase + r)], eidx)
            ws.append(w)
            wsum = wsum + w

        scale = SCALING / (wsum + 1e-20)
        for r in range(TOP_K):
            plsc.store_scatter(ow, [lane8 + (obase + r)], ws[r] * scale)
        return 0

    lax.fori_loop(0, n_tiles, tile_body, 0)
    out_base = wid * tok_per_w * TOP_K
    out_len = tok_per_w * TOP_K
    pltpu.sync_copy(oi, idx_hbm.at[pl.ds(out_base, out_len)])
    pltpu.sync_copy(ow, w_hbm.at[pl.ds(out_base, out_len)])


def kernel(router_logits, correction_bias):
    scores = jax.nn.sigmoid(router_logits).astype(jnp.float32)
    adj = scores + correction_bias[None, :]
    n_tokens = router_logits.shape[0]
    tok_per_w = n_tokens // NW
    mesh = plsc.VectorSubcoreMesh(core_axis_name="c", subcore_axis_name="s",
                                  num_cores=NC, num_subcores=NS)
    run = pl.kernel(
        _router_body,
        out_type=(
            jax.ShapeDtypeStruct((n_tokens * TOP_K,), jnp.int32),
            jax.ShapeDtypeStruct((n_tokens * TOP_K,), jnp.float32),
        ),
        mesh=mesh,
        scratch_types=[
            pltpu.VMEM((2 * TILE_ELEMS,), jnp.float32),
            pltpu.VMEM((N_SUPER * L,), jnp.float32),
            pltpu.VMEM((N_EXPERTS,), jnp.float32),
            pltpu.VMEM((tok_per_w * TOP_K,), jnp.int32),
            pltpu.VMEM((tok_per_w * TOP_K,), jnp.float32),
            pltpu.SemaphoreType.DMA,
        ],
        compiler_params=pltpu.CompilerParams(use_tc_tiling_on_sc=True,
                                             needs_layout_passes=False),
    )
    idx_flat, w_flat = run(adj.reshape(n_tokens * N_EXPERTS), correction_bias)
    return (idx_flat.reshape(n_tokens, TOP_K),
            w_flat.reshape(n_tokens, TOP_K))
